# baseline (speedup 1.0000x reference)
import jax
import jax.numpy as jnp
from jax import lax
from jax.experimental import pallas as pl
from jax.experimental.pallas import tpu as pltpu

K = 1024
D = 64
BETA = 0.25
P = 1024
IB = 8


def _vq_block(x_ref, emb_ref, out_ref, loss_ref, se_ref):
    first = (pl.program_id(0) == 0) & (pl.program_id(1) == 0)
    e = emb_ref[...]

    @pl.when(first)
    def _init():
        se_ref[...] = jnp.sum(e ** 2, axis=1, keepdims=True)
        loss_ref[...] = jnp.zeros_like(loss_ref)

    e2 = e + e
    for sub in range(IB):
        x = x_ref[sub]
        m2 = lax.dot_general(e2, x, (((1,), (0,)), ((), ())),
                             preferred_element_type=jnp.float32)
        sx = jnp.sum(x ** 2, axis=0, keepdims=True)
        dist = sx + se_ref[...] - m2
        minv = jnp.min(dist, axis=0, keepdims=True)
        ids = lax.broadcasted_iota(jnp.int32, (K, P), 0)
        idx = jnp.min(jnp.where(dist == minv, ids, K), axis=0)
        oh = (ids == idx[None, :]).astype(jnp.float32)
        out_ref[sub] = lax.dot_general(e, oh, (((0,), (0,)), ((), ())),
                                       preferred_element_type=jnp.float32)
        loss_ref[...] += jnp.reshape(jnp.sum(minv), (1, 1))


def kernel(latents, embedding_weight):
    b, c, h, w = latents.shape
    n = b * h * w
    cols = latents.reshape(b, c, h * w)
    out_cols, loss = pl.pallas_call(
        _vq_block,
        grid=(b // IB, h * w // P),
        in_specs=[pl.BlockSpec((IB, D, P), lambda i, j: (i, 0, j)),
                  pl.BlockSpec((K, D), lambda i, j: (0, 0))],
        out_specs=[pl.BlockSpec((IB, D, P), lambda i, j: (i, 0, j)),
                   pl.BlockSpec((1, 1), lambda i, j: (0, 0))],
        out_shape=[jax.ShapeDtypeStruct((b, D, h * w), jnp.float32),
                   jax.ShapeDtypeStruct((1, 1), jnp.float32)],
        scratch_shapes=[pltpu.VMEM((K, 1), jnp.float32)],
    )(cols, embedding_weight)
    l = loss[0, 0] / (n * D)
    return (out_cols.reshape(b, c, h, w), l * BETA, l)


# speedup vs baseline: 1.0058x; 1.0058x over previous
import jax
import jax.numpy as jnp
from jax import lax
from jax.experimental import pallas as pl
from jax.experimental.pallas import tpu as pltpu

K = 1024
D = 64
BETA = 0.25
P = 1024
IB = 4


def _vq_block(x_ref, emb_ref, out_ref, loss_ref, se_ref):
    first = (pl.program_id(0) == 0) & (pl.program_id(1) == 0)
    e = emb_ref[...]

    @pl.when(first)
    def _init():
        se_ref[...] = jnp.sum(e ** 2, axis=1, keepdims=True)
        loss_ref[...] = jnp.zeros_like(loss_ref)

    e2 = e + e
    for sub in range(IB):
        x = x_ref[sub]
        m2 = lax.dot_general(e2, x, (((1,), (0,)), ((), ())),
                             preferred_element_type=jnp.float32)
        sx = jnp.sum(x ** 2, axis=0, keepdims=True)
        dist = sx + se_ref[...] - m2
        minv = jnp.min(dist, axis=0, keepdims=True)
        ids = lax.broadcasted_iota(jnp.int32, (K, P), 0)
        idx = jnp.min(jnp.where(dist == minv, ids, K), axis=0)
        oh = (ids == idx[None, :]).astype(jnp.float32)
        out_ref[sub] = lax.dot_general(e, oh, (((0,), (0,)), ((), ())),
                                       preferred_element_type=jnp.float32)
        loss_ref[...] += jnp.reshape(jnp.sum(minv), (1, 1))


def kernel(latents, embedding_weight):
    b, c, h, w = latents.shape
    n = b * h * w
    cols = latents.reshape(b, c, h * w)
    out_cols, loss = pl.pallas_call(
        _vq_block,
        grid=(b // IB, h * w // P),
        in_specs=[pl.BlockSpec((IB, D, P), lambda i, j: (i, 0, j)),
                  pl.BlockSpec((K, D), lambda i, j: (0, 0))],
        out_specs=[pl.BlockSpec((IB, D, P), lambda i, j: (i, 0, j)),
                   pl.BlockSpec((1, 1), lambda i, j: (0, 0))],
        out_shape=[jax.ShapeDtypeStruct((b, D, h * w), jnp.float32),
                   jax.ShapeDtypeStruct((1, 1), jnp.float32)],
        scratch_shapes=[pltpu.VMEM((K, 1), jnp.float32)],
    )(cols, embedding_weight)
    l = loss[0, 0] / (n * D)
    return (out_cols.reshape(b, c, h, w), l * BETA, l)
